# baseline (speedup 1.0000x reference)
import functools

import jax
import jax.numpy as jnp
from jax.experimental import pallas as pl

N_LAYERS = 4
N_GRAPHS = 64
NUM_CLASSES = 55
E_B = 2000


def _edge_mlp_kernel(e1_ref, cd_ref, w2_ref, b2_ref, wc1_ref, bc1_ref,
                     wc2_ref, m_ref, trans_ref):
    silu = jax.nn.silu
    m1 = silu(e1_ref[...])
    m = silu(jnp.dot(m1, w2_ref[...], preferred_element_type=jnp.float32)
             + b2_ref[...])
    q = silu(jnp.dot(m, wc1_ref[...], preferred_element_type=jnp.float32)
             + bc1_ref[...])
    cm = jnp.dot(q, wc2_ref[...], preferred_element_type=jnp.float32)
    m_ref[...] = m
    trans_ref[...] = cd_ref[...] * cm[:, :4]


def _edge_mlp(e1, cd, w2, b2, wc1, bc1, wc2):
    e = e1.shape[0]
    f = e1.shape[1]
    grid = e // E_B
    wc2b = jnp.broadcast_to(wc2, (f, 128))
    m, trans = pl.pallas_call(
        _edge_mlp_kernel,
        grid=(grid,),
        in_specs=[
            pl.BlockSpec((E_B, f), lambda i: (i, 0)),
            pl.BlockSpec((E_B, 4), lambda i: (i, 0)),
            pl.BlockSpec((f, f), lambda i: (0, 0)),
            pl.BlockSpec((f,), lambda i: (0,)),
            pl.BlockSpec((f, f), lambda i: (0, 0)),
            pl.BlockSpec((f,), lambda i: (0,)),
            pl.BlockSpec((f, 128), lambda i: (0, 0)),
        ],
        out_specs=[
            pl.BlockSpec((E_B, f), lambda i: (i, 0)),
            pl.BlockSpec((E_B, 4), lambda i: (i, 0)),
        ],
        out_shape=[
            jax.ShapeDtypeStruct((e, f), jnp.float32),
            jax.ShapeDtypeStruct((e, 4), jnp.float32),
        ],
    )(e1, cd, w2, b2, wc1, bc1, wc2b)
    return m, trans


def kernel(h, x, params, edge_index, batch):
    silu = jax.nn.silu
    n_nodes = h.shape[0]
    row, col = edge_index[0], edge_index[1]
    h = h @ params['emb_in_w'] + params['emb_in_b']
    coord = x
    cnt = jax.ops.segment_sum(jnp.ones((row.shape[0], 1), jnp.float32), row,
                              num_segments=n_nodes)
    inv_cnt = 1.0 / jnp.maximum(cnt, 1.0)
    for i in range(N_LAYERS):
        p = lambda n, i=i: params['l%d_%s' % (i, n)]
        w1 = p('edge_w1')
        pr = h @ w1[:128] + p('edge_b1')
        pc = h @ w1[128:256]
        w1c = w1[256]
        coord_diff = coord[row] - coord[col]
        radial = jnp.sum(coord_diff ** 2, axis=1, keepdims=True)
        e1 = pr[row] + pc[col] + radial * w1c
        cd4 = jnp.pad(coord_diff, ((0, 0), (0, 1)))
        m, trans4 = _edge_mlp(e1, cd4, p('edge_w2'), p('edge_b2'),
                              p('coord_w1'), p('coord_b1'), p('coord_w2'))
        seg_sum = jax.ops.segment_sum(trans4[:, :3], row, num_segments=n_nodes)
        coord = coord + seg_sum * inv_cnt
        agg = jax.ops.segment_sum(m, row, num_segments=n_nodes)
        n_in = jnp.concatenate([h, agg], axis=1)
        hn = silu(n_in @ p('node_w1') + p('node_b1'))
        h = hn @ p('node_w2') + p('node_b2')
    h = h @ params['emb_out_w'] + params['emb_out_b']
    h_pool = jax.ops.segment_max(h, batch, num_segments=N_GRAPHS)
    z = jax.nn.relu(h_pool @ params['fc1_w'] + params['fc1_b'])
    z = jax.nn.relu(z @ params['fc2_w'] + params['fc2_b'])
    logits = z @ params['fc3_w'] + params['fc3_b']
    return jax.nn.log_softmax(logits, axis=-1)


# speedup vs baseline: 1.7381x; 1.7381x over previous
import functools

import jax
import jax.numpy as jnp
from jax import lax
from jax.experimental import pallas as pl
from jax.experimental.pallas import tpu as pltpu
from jax.experimental.pallas import tpu_sc as plsc

N_LAYERS = 4
N_GRAPHS = 64
NUM_CLASSES = 55
HID = 128

NC, NS = 2, 16
NW = NC * NS
CHUNK = 128
NB = 2
E_B = 2048

_mesh = plsc.VectorSubcoreMesh(core_axis_name="c", subcore_axis_name="s")


def _gather_body(pr_hbm, pc_hbm, cx_hbm, cy_hbm, cz_hbm, row_hbm, col_hbm,
                 prr_hbm, pcr_hbm, cd_hbm,
                 rowv, colv, cxv, cyv, czv,
                 cdb0, cdb1, prs0, prs1, pcs0, pcs1,
                 sg_pr, sg_pc, ss_pr, ss_pc, ss_cd):
    cdb = [cdb0, cdb1]
    prs = [prs0, prs1]
    pcs = [pcs0, pcs1]
    k_chunks = rowv.shape[0]
    wid = lax.axis_index("s") * NC + lax.axis_index("c")
    kbase = wid * k_chunks
    ebase = wid * (k_chunks * CHUNK)

    pltpu.sync_copy(row_hbm.at[pl.ds(kbase, k_chunks)], rowv)
    pltpu.sync_copy(col_hbm.at[pl.ds(kbase, k_chunks)], colv)
    pltpu.sync_copy(cx_hbm, cxv)
    pltpu.sync_copy(cy_hbm, cyv)
    pltpu.sync_copy(cz_hbm, czv)

    def issue_gathers(k, b):
        pltpu.async_copy(pr_hbm.at[rowv.at[k]], prs[b], sg_pr.at[b])
        pltpu.async_copy(pc_hbm.at[colv.at[k]], pcs[b], sg_pc.at[b])

    def wait_gathers(k, b):
        pltpu.make_async_copy(pr_hbm.at[rowv.at[k]], prs[b], sg_pr.at[b]).wait()
        pltpu.make_async_copy(pc_hbm.at[colv.at[k]], pcs[b], sg_pc.at[b]).wait()

    def _st(k, b):
        sl = pl.ds(ebase + k * CHUNK, CHUNK)
        return ((prs[b], prr_hbm.at[sl], ss_pr.at[b]),
                (pcs[b], pcr_hbm.at[sl], ss_pc.at[b]),
                (cdb[b], cd_hbm.at[pl.ds((ebase + k * CHUNK) * 16,
                                            CHUNK * 16)], ss_cd.at[b]))

    def issue_stores(k, b):
        for src, dst, sem in _st(k, b):
            pltpu.async_copy(src, dst, sem)

    def wait_stores(k, b):
        for src, dst, sem in _st(k, b):
            pltpu.make_async_copy(src, dst, sem).wait()

    lane16 = lax.iota(jnp.int32, 16)

    def compute_cd(k, b):
        for g in range(CHUNK // 16):
            r16 = rowv[k, pl.ds(g * 16, 16)]
            c16 = colv[k, pl.ds(g * 16, 16)]
            dx = plsc.load_gather(cxv, [r16]) - plsc.load_gather(cxv, [c16])
            dy = plsc.load_gather(cyv, [r16]) - plsc.load_gather(cyv, [c16])
            dz = plsc.load_gather(czv, [r16]) - plsc.load_gather(czv, [c16])
            rad = dx * dx + dy * dy + dz * dz
            pos = g * 256 + lane16 * 16
            plsc.store_scatter(cdb[b], [pos], dx)
            plsc.store_scatter(cdb[b], [pos + 1], dy)
            plsc.store_scatter(cdb[b], [pos + 2], dz)
            plsc.store_scatter(cdb[b], [pos + 3], rad)

    for b in range(NB - 1):
        issue_gathers(b, b)

    def body(i, carry):
        for b in range(NB):
            k = i * NB + b
            bi = (b + NB - 1) % NB
            kn = k + NB - 1

            @pl.when(k >= 1)
            def _():
                wait_stores(k - 1, bi)

            @pl.when(kn < k_chunks)
            def _():
                issue_gathers(kn, bi)

            compute_cd(k, b)
            wait_gathers(k, b)
            issue_stores(k, b)
        return carry

    lax.fori_loop(0, k_chunks // NB, body, 0)
    wait_stores(k_chunks - 1, NB - 1)


def _sc_gather(pr, pc, cx, cy, cz, row2d, col2d, ep):
    k_chunks = ep // (NW * CHUNK)
    n = pr.shape[0]
    f32 = jnp.float32
    kern = pl.kernel(
        _gather_body,
        out_type=[
            jax.ShapeDtypeStruct((ep, HID), f32),
            jax.ShapeDtypeStruct((ep, HID), f32),
            jax.ShapeDtypeStruct((ep * 16,), f32),
        ],
        mesh=_mesh,
        scratch_types=[
            pltpu.VMEM((k_chunks, CHUNK), jnp.int32),
            pltpu.VMEM((k_chunks, CHUNK), jnp.int32),
            pltpu.VMEM((n,), f32),
            pltpu.VMEM((n,), f32),
            pltpu.VMEM((n,), f32),
        ] + [pltpu.VMEM((CHUNK * 16,), f32)] * NB
          + [pltpu.VMEM((CHUNK, HID), f32)] * (2 * NB)
          + [pltpu.SemaphoreType.DMA((NB,))] * 5,
        compiler_params=pltpu.CompilerParams(needs_layout_passes=False),
    )
    return kern(pr, pc, cx, cy, cz, row2d, col2d)


def _edge_mlp_kernel(prr_ref, pcr_ref, cd_ref, w1c_ref, w2_ref,
                     b2_ref, wc1_ref, bc1_ref, wc2_ref, m_ref, trans_ref):
    silu = jax.nn.silu
    cd = cd_ref[...]
    radial = cd[:, 3:4]
    e1 = prr_ref[...] + pcr_ref[...] + radial * w1c_ref[...]
    m1 = silu(e1)
    m = silu(jnp.dot(m1, w2_ref[...], preferred_element_type=jnp.float32)
             + b2_ref[...])
    q = silu(jnp.dot(m, wc1_ref[...], preferred_element_type=jnp.float32)
             + bc1_ref[...])
    cm = jnp.dot(q, wc2_ref[...], preferred_element_type=jnp.float32)
    m_ref[...] = m
    trans_ref[...] = cd * cm[:, :1]


def _edge_mlp(prr, pcr, cd16, w1c, w2, b2, wc1, bc1, wc2):
    ep = prr.shape[0]
    grid = ep // E_B
    wc2b = jnp.broadcast_to(wc2, (HID, 128))
    m, trans = pl.pallas_call(
        _edge_mlp_kernel,
        grid=(grid,),
        in_specs=[
            pl.BlockSpec((E_B, HID), lambda i: (i, 0)),
            pl.BlockSpec((E_B, HID), lambda i: (i, 0)),
            pl.BlockSpec((E_B, 16), lambda i: (i, 0)),
            pl.BlockSpec((1, HID), lambda i: (0, 0)),
            pl.BlockSpec((HID, HID), lambda i: (0, 0)),
            pl.BlockSpec((HID,), lambda i: (0,)),
            pl.BlockSpec((HID, HID), lambda i: (0, 0)),
            pl.BlockSpec((HID,), lambda i: (0,)),
            pl.BlockSpec((HID, 128), lambda i: (0, 0)),
        ],
        out_specs=[
            pl.BlockSpec((E_B, HID), lambda i: (i, 0)),
            pl.BlockSpec((E_B, 16), lambda i: (i, 0)),
        ],
        out_shape=[
            jax.ShapeDtypeStruct((ep, HID), jnp.float32),
            jax.ShapeDtypeStruct((ep, 16), jnp.float32),
        ],
    )(prr, pcr, cd16, w1c.reshape(1, HID), w2, b2, wc1, bc1, wc2b)
    return m, trans


def kernel(h, x, params, edge_index, batch):
    silu = jax.nn.silu
    n_nodes = h.shape[0]
    e = edge_index.shape[1]
    row, col = edge_index[0], edge_index[1]
    epq = NW * CHUNK * 8
    ep = ((e + epq - 1) // epq) * epq
    pad = ep - e
    row_p = jnp.concatenate([row, jnp.zeros((pad,), jnp.int32)])
    col_p = jnp.concatenate([col, jnp.zeros((pad,), jnp.int32)])
    row2d = row_p.reshape(-1, CHUNK)
    col2d = col_p.reshape(-1, CHUNK)
    row_scat = jnp.concatenate([row, jnp.full((pad,), n_nodes, jnp.int32)])

    h = h @ params['emb_in_w'] + params['emb_in_b']
    coord = x
    cnt = jax.ops.segment_sum(jnp.ones((e, 1), jnp.float32), row,
                              num_segments=n_nodes)
    inv_cnt = 1.0 / jnp.maximum(cnt, 1.0)
    for i in range(N_LAYERS):
        p = lambda n, i=i: params['l%d_%s' % (i, n)]
        w1 = p('edge_w1')
        pr = h @ w1[:HID] + p('edge_b1')
        pc = h @ w1[HID:2 * HID]
        prr, pcr, cdflat = _sc_gather(pr, pc, coord[:, 0], coord[:, 1],
                                      coord[:, 2], row2d, col2d, ep)
        cd16 = cdflat.reshape(ep, 16)
        m, trans16 = _edge_mlp(prr, pcr, cd16, w1[2 * HID],
                               p('edge_w2'), p('edge_b2'),
                               p('coord_w1'), p('coord_b1'), p('coord_w2'))
        tsum = jax.ops.segment_sum(trans16[:, :3], row_scat,
                                   num_segments=n_nodes)
        coord = coord + tsum * inv_cnt
        agg = jax.ops.segment_sum(m, row_scat, num_segments=n_nodes)
        n_in = jnp.concatenate([h, agg], axis=1)
        hn = silu(n_in @ p('node_w1') + p('node_b1'))
        h = hn @ p('node_w2') + p('node_b2')
    h = h @ params['emb_out_w'] + params['emb_out_b']
    h_pool = jax.ops.segment_max(h, batch, num_segments=N_GRAPHS)
    z = jax.nn.relu(h_pool @ params['fc1_w'] + params['fc1_b'])
    z = jax.nn.relu(z @ params['fc2_w'] + params['fc2_b'])
    logits = z @ params['fc3_w'] + params['fc3_b']
    return jax.nn.log_softmax(logits, axis=-1)


# speedup vs baseline: 3.4804x; 2.0024x over previous
import functools

import jax
import jax.numpy as jnp
from jax import lax
from jax.experimental import pallas as pl
from jax.experimental.pallas import tpu as pltpu
from jax.experimental.pallas import tpu_sc as plsc

N_LAYERS = 4
N_GRAPHS = 64
NUM_CLASSES = 55
HID = 128

NC, NS = 2, 16
NW = NC * NS
CHUNK = 128
NB = 2
E_B = 2048

_mesh = plsc.VectorSubcoreMesh(core_axis_name="c", subcore_axis_name="s")


def _gather_body(pr_hbm, pc_hbm, cx_hbm, cy_hbm, cz_hbm, row_hbm, col_hbm,
                 prr_hbm, pcr_hbm, cd_hbm,
                 rowv, colv, cxv, cyv, czv,
                 cdb0, cdb1, prs0, prs1, pcs0, pcs1,
                 sg_pr, sg_pc, ss_pr, ss_pc, ss_cd):
    cdb = [cdb0, cdb1]
    prs = [prs0, prs1]
    pcs = [pcs0, pcs1]
    k_chunks = rowv.shape[0]
    wid = lax.axis_index("s") * NC + lax.axis_index("c")
    kbase = wid * k_chunks
    ebase = wid * (k_chunks * CHUNK)

    pltpu.sync_copy(row_hbm.at[pl.ds(kbase, k_chunks)], rowv)
    pltpu.sync_copy(col_hbm.at[pl.ds(kbase, k_chunks)], colv)
    pltpu.sync_copy(cx_hbm, cxv)
    pltpu.sync_copy(cy_hbm, cyv)
    pltpu.sync_copy(cz_hbm, czv)

    def issue_gathers(k, b):
        pltpu.async_copy(pr_hbm.at[rowv.at[k]], prs[b], sg_pr.at[b])
        pltpu.async_copy(pc_hbm.at[colv.at[k]], pcs[b], sg_pc.at[b])

    def wait_gathers(k, b):
        pltpu.make_async_copy(pr_hbm.at[rowv.at[k]], prs[b], sg_pr.at[b]).wait()
        pltpu.make_async_copy(pc_hbm.at[colv.at[k]], pcs[b], sg_pc.at[b]).wait()

    def _st(k, b):
        sl = pl.ds(ebase + k * CHUNK, CHUNK)
        return ((prs[b], prr_hbm.at[sl], ss_pr.at[b]),
                (pcs[b], pcr_hbm.at[sl], ss_pc.at[b]),
                (cdb[b], cd_hbm.at[pl.ds((ebase + k * CHUNK) * 16,
                                            CHUNK * 16)], ss_cd.at[b]))

    def issue_stores(k, b):
        for src, dst, sem in _st(k, b):
            pltpu.async_copy(src, dst, sem)

    def wait_stores(k, b):
        for src, dst, sem in _st(k, b):
            pltpu.make_async_copy(src, dst, sem).wait()

    lane16 = lax.iota(jnp.int32, 16)

    def compute_cd(k, b):
        for g in range(CHUNK // 16):
            r16 = rowv[k, pl.ds(g * 16, 16)]
            c16 = colv[k, pl.ds(g * 16, 16)]
            dx = plsc.load_gather(cxv, [r16]) - plsc.load_gather(cxv, [c16])
            dy = plsc.load_gather(cyv, [r16]) - plsc.load_gather(cyv, [c16])
            dz = plsc.load_gather(czv, [r16]) - plsc.load_gather(czv, [c16])
            rad = dx * dx + dy * dy + dz * dz
            pos = g * 256 + lane16 * 16
            plsc.store_scatter(cdb[b], [pos], dx)
            plsc.store_scatter(cdb[b], [pos + 1], dy)
            plsc.store_scatter(cdb[b], [pos + 2], dz)
            plsc.store_scatter(cdb[b], [pos + 3], rad)

    for b in range(NB - 1):
        issue_gathers(b, b)

    def body(i, carry):
        for b in range(NB):
            k = i * NB + b
            bi = (b + NB - 1) % NB
            kn = k + NB - 1

            @pl.when(k >= 1)
            def _():
                wait_stores(k - 1, bi)

            @pl.when(kn < k_chunks)
            def _():
                issue_gathers(kn, bi)

            compute_cd(k, b)
            wait_gathers(k, b)
            issue_stores(k, b)
        return carry

    lax.fori_loop(0, k_chunks // NB, body, 0)
    wait_stores(k_chunks - 1, NB - 1)


def _sc_gather(pr, pc, cx, cy, cz, row2d, col2d, ep):
    k_chunks = ep // (NW * CHUNK)
    n = pr.shape[0]
    f32 = jnp.float32
    kern = pl.kernel(
        _gather_body,
        out_type=[
            jax.ShapeDtypeStruct((ep, HID), f32),
            jax.ShapeDtypeStruct((ep, HID), f32),
            jax.ShapeDtypeStruct((ep * 16,), f32),
        ],
        mesh=_mesh,
        scratch_types=[
            pltpu.VMEM((k_chunks, CHUNK), jnp.int32),
            pltpu.VMEM((k_chunks, CHUNK), jnp.int32),
            pltpu.VMEM((n,), f32),
            pltpu.VMEM((n,), f32),
            pltpu.VMEM((n,), f32),
        ] + [pltpu.VMEM((CHUNK * 16,), f32)] * NB
          + [pltpu.VMEM((CHUNK, HID), f32)] * (2 * NB)
          + [pltpu.SemaphoreType.DMA((NB,))] * 5,
        compiler_params=pltpu.CompilerParams(needs_layout_passes=False),
    )
    return kern(pr, pc, cx, cy, cz, row2d, col2d)


NPAD = 10112
KT = 80
_STRIPE_CHUNKS = [(0, 128), (128, 128), (256, 128), (384, 128), (512, 120)]


def _scatter_body(m_hbm, cm_hbm, cd_hbm, idx_hbm, zm_hbm, zt_hbm,
                  om_hbm, ot_hbm,
                  ix0, ix1, ms0, ms1, ts0, cds0, cds1, cms0, cms1,
                  accm, acct,
                  sgi, sgm, sgd, sgc, ssm, sst):
    ix = [ix0, ix1]
    ms = [ms0, ms1]
    cds = [cds0, cds1]
    cms = [cms0, cms1]
    sid = lax.axis_index("s")
    cid = lax.axis_index("c")
    wid = sid * NC + cid
    ebase = wid * (KT * CHUNK)
    nstripe = NPAD // NS

    for off, ln in _STRIPE_CHUNKS:
        sl = pl.ds(sid * nstripe + off, ln)
        pltpu.sync_copy(zm_hbm.at[pl.ds(0, ln)], accm.at[sl])
        pltpu.sync_copy(zt_hbm.at[pl.ds(0, ln)], acct.at[sl])
    plsc.subcore_barrier()

    def _ld(k, b):
        sl = pl.ds(ebase + k * CHUNK, CHUNK)
        return ((idx_hbm.at[wid * KT + k], ix[b], sgi.at[b]),
                (m_hbm.at[sl], ms[b], sgm.at[b]),
                (cd_hbm.at[pl.ds((ebase + k * CHUNK) * 16, CHUNK * 16)],
                 cds[b], sgd.at[b]),
                (cm_hbm.at[sl], cms[b], sgc.at[b]))

    def _scm(b):
        return (ms[b], accm.at[ix[b]], ssm.at[b])

    def _sct(b):
        return (ts0, acct.at[ix[b]], sst.at[0])

    lane16 = lax.iota(jnp.int32, 16)
    ones16 = jnp.ones((16,), jnp.float32)
    c0 = lane16 * 0
    c1 = c0 + 1
    c2 = c0 + 2
    c3 = c0 + 3

    def compute_t(b):
        for g in range(CHUNK // 16):
            cmv = cms[b][pl.ds(g * 16, 16)]
            pos = (g * 16 + lane16) * 16
            dx = plsc.load_gather(cds[b], [pos])
            dy = plsc.load_gather(cds[b], [pos + 1])
            dz = plsc.load_gather(cds[b], [pos + 2])
            ridx = g * 16 + lane16
            plsc.store_scatter(ts0, [ridx, c0], dx * cmv)
            plsc.store_scatter(ts0, [ridx, c1], dy * cmv)
            plsc.store_scatter(ts0, [ridx, c2], dz * cmv)
            plsc.store_scatter(ts0, [ridx, c3], ones16)

    def body(i, carry):
        for b in range(2):
            k = i * 2 + b

            @pl.when(i > 0)
            def _():
                s, d, sem = _scm(b)
                pltpu.make_async_copy(s, d, sem).wait()

            for s, d, sem in _ld(k, b):
                pltpu.async_copy(s, d, sem)
        for b in range(2):
            k = i * 2 + b
            for s, d, sem in _ld(k, b):
                pltpu.make_async_copy(s, d, sem).wait()

            @pl.when(k > 0)
            def _():
                s, d, sem = _sct(b)
                pltpu.make_async_copy(s, d, sem).wait()

            compute_t(b)
            for s, d, sem in (_scm(b), _sct(b)):
                pltpu.async_copy(s, d, sem, add=True)
        return carry

    lax.fori_loop(0, KT // 2, body, 0)
    for b in range(2):
        s, d, sem = _scm(b)
        pltpu.make_async_copy(s, d, sem).wait()
    s, d, sem = _sct(0)
    pltpu.make_async_copy(s, d, sem).wait()
    plsc.subcore_barrier()
    for off, ln in _STRIPE_CHUNKS:
        sl = pl.ds(sid * nstripe + off, ln)
        pltpu.sync_copy(accm.at[sl], om_hbm.at[cid].at[sl])
        pltpu.sync_copy(acct.at[sl], ot_hbm.at[cid].at[sl])


def _sc_scatter(m, cmflat, cdflat, idx2d):
    f32 = jnp.float32
    zm = jnp.zeros((CHUNK, HID), f32)
    zt = jnp.zeros((CHUNK, 16), f32)
    kern = pl.kernel(
        _scatter_body,
        out_type=[
            jax.ShapeDtypeStruct((NC, NPAD, HID), f32),
            jax.ShapeDtypeStruct((NC, NPAD, 16), f32),
        ],
        mesh=_mesh,
        scratch_types=[
            pltpu.VMEM((CHUNK,), jnp.int32),
            pltpu.VMEM((CHUNK,), jnp.int32),
            pltpu.VMEM((CHUNK, HID), f32),
            pltpu.VMEM((CHUNK, HID), f32),
            pltpu.VMEM((CHUNK, 16), f32),
            pltpu.VMEM((CHUNK * 16,), f32),
            pltpu.VMEM((CHUNK * 16,), f32),
            pltpu.VMEM((CHUNK,), f32),
            pltpu.VMEM((CHUNK,), f32),
            pltpu.VMEM_SHARED((NPAD, HID), f32),
            pltpu.VMEM_SHARED((NPAD, 16), f32),
        ] + [pltpu.SemaphoreType.DMA((2,))] * 6,
        compiler_params=pltpu.CompilerParams(
            needs_layout_passes=False, use_tc_tiling_on_sc=False),
    )
    return kern(m, cmflat, cdflat, idx2d, zm, zt)


def _edge_mlp_kernel(prr_ref, pcr_ref, cd_ref, w1c_ref, w2_ref,
                     b2_ref, wc1_ref, bc1_ref, wc2_ref, m_ref, trans_ref):
    silu = jax.nn.silu
    cd = cd_ref[...]
    radial = cd[:, 3:4]
    e1 = prr_ref[...] + pcr_ref[...] + radial * w1c_ref[...]
    m1 = silu(e1)
    m = silu(jnp.dot(m1, w2_ref[...], preferred_element_type=jnp.float32)
             + b2_ref[...])
    q = silu(jnp.dot(m, wc1_ref[...], preferred_element_type=jnp.float32)
             + bc1_ref[...])
    cm = jnp.dot(q, wc2_ref[...], preferred_element_type=jnp.float32)
    m_ref[...] = m
    trans_ref[...] = cm[:, :1].reshape(cd.shape[0] // 128, 128)


def _edge_mlp(prr, pcr, cd16, w1c, w2, b2, wc1, bc1, wc2):
    ep = prr.shape[0]
    grid = ep // E_B
    wc2b = jnp.broadcast_to(wc2, (HID, 128))
    m, trans = pl.pallas_call(
        _edge_mlp_kernel,
        grid=(grid,),
        in_specs=[
            pl.BlockSpec((E_B, HID), lambda i: (i, 0)),
            pl.BlockSpec((E_B, HID), lambda i: (i, 0)),
            pl.BlockSpec((E_B, 16), lambda i: (i, 0)),
            pl.BlockSpec((1, HID), lambda i: (0, 0)),
            pl.BlockSpec((HID, HID), lambda i: (0, 0)),
            pl.BlockSpec((HID,), lambda i: (0,)),
            pl.BlockSpec((HID, HID), lambda i: (0, 0)),
            pl.BlockSpec((HID,), lambda i: (0,)),
            pl.BlockSpec((HID, 128), lambda i: (0, 0)),
        ],
        out_specs=[
            pl.BlockSpec((E_B, HID), lambda i: (i, 0)),
            pl.BlockSpec((E_B // 128, 128), lambda i: (i, 0)),
        ],
        out_shape=[
            jax.ShapeDtypeStruct((ep, HID), jnp.float32),
            jax.ShapeDtypeStruct((ep // 128, 128), jnp.float32),
        ],
    )(prr, pcr, cd16, w1c.reshape(1, HID), w2, b2, wc1, bc1, wc2b)
    return m, trans


def kernel(h, x, params, edge_index, batch):
    silu = jax.nn.silu
    n_nodes = h.shape[0]
    e = edge_index.shape[1]
    row, col = edge_index[0], edge_index[1]
    epq = NW * CHUNK * 8
    ep = ((e + epq - 1) // epq) * epq
    pad = ep - e
    row_p = jnp.concatenate([row, jnp.zeros((pad,), jnp.int32)])
    col_p = jnp.concatenate([col, jnp.zeros((pad,), jnp.int32)])
    row2d = row_p.reshape(-1, CHUNK)
    col2d = col_p.reshape(-1, CHUNK)
    rowscat2d = jnp.concatenate(
        [row, jnp.full((pad,), n_nodes, jnp.int32)]).reshape(-1, CHUNK)

    h = h @ params['emb_in_w'] + params['emb_in_b']
    coord = x
    for i in range(N_LAYERS):
        p = lambda n, i=i: params['l%d_%s' % (i, n)]
        w1 = p('edge_w1')
        pr = h @ w1[:HID] + p('edge_b1')
        pc = h @ w1[HID:2 * HID]
        prr, pcr, cdflat = _sc_gather(pr, pc, coord[:, 0], coord[:, 1],
                                      coord[:, 2], row2d, col2d, ep)
        cd16 = cdflat.reshape(ep, 16)
        m, cmpk = _edge_mlp(prr, pcr, cd16, w1[2 * HID],
                            p('edge_w2'), p('edge_b2'),
                            p('coord_w1'), p('coord_b1'), p('coord_w2'))
        om, ot = _sc_scatter(m, cmpk.reshape(ep), cdflat, rowscat2d)
        agg = om[0, :n_nodes] + om[1, :n_nodes]
        tacc = ot[0, :n_nodes] + ot[1, :n_nodes]
        cnt = tacc[:, 3:4]
        coord = coord + tacc[:, :3] / jnp.maximum(cnt, 1.0)
        n_in = jnp.concatenate([h, agg], axis=1)
        hn = silu(n_in @ p('node_w1') + p('node_b1'))
        h = hn @ p('node_w2') + p('node_b2')
    h = h @ params['emb_out_w'] + params['emb_out_b']
    h_pool = jax.ops.segment_max(h, batch, num_segments=N_GRAPHS)
    z = jax.nn.relu(h_pool @ params['fc1_w'] + params['fc1_b'])
    z = jax.nn.relu(z @ params['fc2_w'] + params['fc2_b'])
    logits = z @ params['fc3_w'] + params['fc3_b']
    return jax.nn.log_softmax(logits, axis=-1)
